# baseline (speedup 1.0000x reference)
import jax
import jax.numpy as jnp
from jax import lax
from jax.experimental import pallas as pl
from jax.experimental.pallas import tpu as pltpu

INPUT_DIM = 2048
K = 512
CURRENT_TEMP = 5.0
ROWS_PER_BLOCK = 512
ONE_BITS = 0x3F800000


def _body(soft_ref, x_ref, out_ref, mask_ref):
    @pl.when(pl.program_id(0) == 0)
    def _compute_mask():
        soft = soft_ref[...]
        bits = lax.bitcast_convert_type(soft, jnp.int32)

        def step(_, carry):
            lo, hi = carry
            mid = lo + (hi - lo + 1) // 2
            cnt = jnp.sum((bits >= mid).astype(jnp.int32))
            pred = cnt >= K
            return jnp.where(pred, mid, lo), jnp.where(pred, hi, mid - 1)

        lo, _ = lax.fori_loop(
            0, 31, step, (jnp.int32(0), jnp.int32(ONE_BITS)))
        mask_ref[...] = (bits >= lo).astype(jnp.float32)

    out_ref[...] = x_ref[...] * mask_ref[...]


def kernel(x, logits):
    noise_key = jax.random.key(42)
    u = jax.random.uniform(noise_key, logits.shape, dtype=logits.dtype)
    noise = -jnp.log(-jnp.log(u + 1e-20) + 1e-20)
    soft_probs = jax.nn.sigmoid((logits + noise) / CURRENT_TEMP)
    soft2d = soft_probs.reshape(1, INPUT_DIM)

    b, s, d = x.shape
    rows = b * s
    x2 = x.reshape(rows, d)
    grid = rows // ROWS_PER_BLOCK

    out = pl.pallas_call(
        _body,
        grid=(grid,),
        in_specs=[
            pl.BlockSpec((1, d), lambda i: (0, 0)),
            pl.BlockSpec((ROWS_PER_BLOCK, d), lambda i: (i, 0)),
        ],
        out_specs=pl.BlockSpec((ROWS_PER_BLOCK, d), lambda i: (i, 0)),
        out_shape=jax.ShapeDtypeStruct((rows, d), x.dtype),
        scratch_shapes=[pltpu.VMEM((1, d), jnp.float32)],
    )(soft2d, x2)
    return out.reshape(b, s, d)
